# baseline (speedup 1.0000x reference)
import functools

import jax
import jax.numpy as jnp
from jax import lax
from jax.experimental import pallas as pl
from jax.experimental.pallas import tpu as pltpu
from jax.experimental.pallas import tpu_sc as plsc

RANK = 32
BATCH = 16384
LANES = 16
NC = 2
NS = 16
NW = NC * NS
BPW = BATCH // NW
CH = 128
NCH = BPW // CH
GROUPS = BPW // LANES

_mesh = plsc.VectorSubcoreMesh(core_axis_name="c", subcore_axis_name="s")


@functools.partial(
    pl.kernel,
    mesh=_mesh,
    compiler_params=pltpu.CompilerParams(
        needs_layout_passes=False, use_tc_tiling_on_sc=False),
    out_type=jax.ShapeDtypeStruct((BATCH,), jnp.float32),
    scratch_types=[
        pltpu.VMEM((NCH, CH), jnp.int32),
        pltpu.VMEM((NCH, CH), jnp.int32),
        pltpu.VMEM((NCH, CH), jnp.int32),
        pltpu.VMEM((BPW, RANK), jnp.float32),
        pltpu.VMEM((BPW, RANK), jnp.float32),
        pltpu.VMEM((BPW, RANK), jnp.float32),
        pltpu.VMEM((BPW,), jnp.float32),
        pltpu.SemaphoreType.DMA,
    ],
)
def _hungrey_sc(t_idx_hbm, u_idx_hbm, s_idx_hbm, u_tab, s_tab, t_tab,
                out_hbm, t_idx_v, u_idx_v, s_idx_v, u_rows, s_rows, t_rows,
                out_v, sem):
    wid = lax.axis_index("s") * NC + lax.axis_index("c")
    ibase = wid * NCH

    pltpu.sync_copy(t_idx_hbm.at[pl.ds(ibase, NCH)], t_idx_v)
    pltpu.sync_copy(u_idx_hbm.at[pl.ds(ibase, NCH)], u_idx_v)
    pltpu.sync_copy(s_idx_hbm.at[pl.ds(ibase, NCH)], s_idx_v)

    copies = []
    for j in range(NCH):
        dst = pl.ds(j * CH, CH)
        copies.append(pltpu.make_async_copy(u_tab.at[u_idx_v.at[j]],
                                            u_rows.at[dst], sem))
        copies.append(pltpu.make_async_copy(s_tab.at[s_idx_v.at[j]],
                                            s_rows.at[dst], sem))
        copies.append(pltpu.make_async_copy(t_tab.at[t_idx_v.at[j]],
                                            t_rows.at[dst], sem))
    for c in copies:
        c.start()
    for c in copies:
        c.wait()

    lo = pl.ds(0, LANES)
    hi = pl.ds(LANES, LANES)
    lane = lax.iota(jnp.int32, LANES)
    eq = [lane == k for k in range(LANES)]

    def group(g, carry):
        acc = jnp.zeros((LANES,), jnp.float32)
        for k in range(LANES):
            r = g * LANES + k
            p = (u_rows[r, lo] * s_rows[r, lo] * t_rows[r, lo]
                 + u_rows[r, hi] * s_rows[r, hi] * t_rows[r, hi])
            sv = jnp.broadcast_to(jnp.sum(p), (LANES,))
            acc = jnp.where(eq[k], sv, acc)
        y = 1.0 / (1.0 + jnp.exp(-acc))
        out_v[pl.ds(g * LANES, LANES)] = y
        return carry

    lax.fori_loop(0, GROUPS, group, 0)

    pltpu.sync_copy(out_v, out_hbm.at[pl.ds(wid * BPW, BPW)])


def kernel(timeIdx, userIdx, servIdx, userEmb, servEmb, timeEmb):
    t_idx = timeIdx.astype(jnp.int32).reshape(BATCH // CH, CH)
    u_idx = userIdx.astype(jnp.int32).reshape(BATCH // CH, CH)
    s_idx = servIdx.astype(jnp.int32).reshape(BATCH // CH, CH)
    return _hungrey_sc(t_idx, u_idx, s_idx, userEmb, servEmb, timeEmb)
